# baseline (speedup 1.0000x reference)
import jax
import jax.numpy as jnp
from jax import lax
from jax.experimental import pallas as pl
from jax.experimental.pallas import tpu as pltpu
from jax.experimental.pallas import tpu_sc as plsc

_N = 10000
_E = 160000
_D = 256
_HALF = 128
_TILES = 16
_K = 128
_NB = 80
_EPAD = _TILES * _NB * _K
_NPAD = 10240
_RPT = _NPAD // _TILES
_RCH = 80
_DPT = _NPAD // _TILES


def _sc_aggregate(x_lo, x_hi, src3, dst3):
    mesh = plsc.VectorSubcoreMesh(core_axis_name="c", subcore_axis_name="s")

    def body(x0, x1, s4, d4, agg0, agg1, deg,
             agg_sp, deg_sp, rows0, rows1, si0, si1, di0, di1, hist, midx,
             dbuf, sem0, sem1, isem0, isem1):
        c = lax.axis_index("c")
        s = lax.axis_index("s")
        ones16 = jnp.full((16,), 1.0, jnp.float32)
        zeros16 = jnp.zeros((16,), jnp.float32)
        iota16 = lax.iota(jnp.int32, 16)

        def hist_batch(di):
            for u in range(_K // 16):
                d = di[pl.ds(u * 16, 16)]
                plsc.addupdate_scatter(hist, [d >> 7, d & 127], ones16)

        def run(xc, aggc, do_deg):
            nch = jnp.where(s == _TILES - 1, 5, _RPT // _RCH)
            if do_deg:
                def zhist(i, carry):
                    for u in range(8):
                        hist[i, pl.ds(u * 16, 16)] = zeros16
                    return carry

                lax.fori_loop(0, _NPAD // 128, zhist, 0)
                for i in range(5):
                    midx[pl.ds(i * 16, 16)] = i * 16 + iota16

                @pl.when(s < 10)
                def _():
                    pltpu.sync_copy(hist.at[pl.ds(0, 8)],
                                    deg_sp.at[pl.ds(s * 8, 8)])

            stage = rows0.at[pl.ds(0, _RCH)]

            def init_chunk(j, carry):
                base = s * _RPT + j * _RCH
                pltpu.sync_copy(xc.at[pl.ds(base, _RCH)], stage)
                pltpu.sync_copy(stage, agg_sp.at[pl.ds(base, _RCH)])
                return carry

            lax.fori_loop(0, nch, init_chunk, 0)
            plsc.subcore_barrier()

            def fetch_idx(jj, si, di, isem):
                pltpu.async_copy(s4.at[s, jj, 0], si, isem)
                pltpu.async_copy(d4.at[s, jj, 0], di, isem)

            def wait_idx(jj, si, di, isem):
                pltpu.make_async_copy(s4.at[s, jj, 0], si, isem).wait()
                pltpu.make_async_copy(d4.at[s, jj, 0], di, isem).wait()

            pltpu.sync_copy(s4.at[s, 0, 0], si0)
            pltpu.sync_copy(d4.at[s, 0, 0], di0)
            pltpu.async_copy(xc.at[si0], rows0, sem0)
            fetch_idx(1, si1, di1, isem1)

            def step(h, carry):
                j0 = 2 * h
                j1 = j0 + 1
                wait_idx(j1, si1, di1, isem1)
                pltpu.async_copy(xc.at[si1], rows1, sem1)
                pltpu.make_async_copy(xc.at[si0], rows0, sem0).wait()
                pltpu.sync_copy(rows0, agg_sp.at[di0], add=True)
                if do_deg:
                    hist_batch(di0)

                @pl.when(h + 1 < _NB // 2)
                def _():
                    fetch_idx(j0 + 2, si0, di0, isem0)

                pltpu.make_async_copy(xc.at[si1], rows1, sem1).wait()
                pltpu.sync_copy(rows1, agg_sp.at[di1], add=True)
                if do_deg:
                    hist_batch(di1)

                @pl.when(h + 1 < _NB // 2)
                def _():
                    fetch_idx(j1 + 2, si1, di1, isem1)
                    wait_idx(j0 + 2, si0, di0, isem0)
                    pltpu.async_copy(xc.at[si0], rows0, sem0)

                return carry

            lax.fori_loop(0, _NB // 2, step, 0)

            if do_deg:
                pltpu.sync_copy(hist, deg_sp.at[midx], add=True)
            plsc.subcore_barrier()

            def out_chunk(j, carry):
                base = s * _RPT + j * _RCH
                pltpu.sync_copy(agg_sp.at[pl.ds(base, _RCH)], stage)
                pltpu.sync_copy(stage, aggc.at[pl.ds(base, _RCH)])
                return carry

            lax.fori_loop(0, nch, out_chunk, 0)
            if do_deg:
                @pl.when(s < 10)
                def _():
                    pltpu.sync_copy(deg_sp.at[pl.ds(s * 8, 8)], dbuf)
                    pltpu.sync_copy(dbuf, deg.at[pl.ds(s * 8, 8)])

        @pl.when(c == 0)
        def _():
            run(x0, agg0, True)

        @pl.when(c == 1)
        def _():
            run(x1, agg1, False)

    f = pl.kernel(
        body,
        out_type=[
            jax.ShapeDtypeStruct((_NPAD, _HALF), jnp.float32),
            jax.ShapeDtypeStruct((_NPAD, _HALF), jnp.float32),
            jax.ShapeDtypeStruct((_NPAD // 128, 128), jnp.float32),
        ],
        mesh=mesh,
        compiler_params=pltpu.CompilerParams(needs_layout_passes=False),
        scratch_types=[
            pltpu.VMEM_SHARED((_NPAD, _HALF), jnp.float32),
            pltpu.VMEM_SHARED((_NPAD // 128, 128), jnp.float32),
            pltpu.VMEM((_K, _HALF), jnp.float32),
            pltpu.VMEM((_K, _HALF), jnp.float32),
            pltpu.VMEM((128,), jnp.int32),
            pltpu.VMEM((128,), jnp.int32),
            pltpu.VMEM((128,), jnp.int32),
            pltpu.VMEM((128,), jnp.int32),
            pltpu.VMEM((_NPAD // 128, 128), jnp.float32),
            pltpu.VMEM((80,), jnp.int32),
            pltpu.VMEM((8, 128), jnp.float32),
            pltpu.SemaphoreType.DMA,
            pltpu.SemaphoreType.DMA,
            pltpu.SemaphoreType.DMA,
            pltpu.SemaphoreType.DMA,
        ],
    )
    return f(x_lo, x_hi, src3, dst3)


_BM = 2000


def _tc_body(x_ref, a0_ref, a1_ref, deg_ref, ws_ref, wn0_ref, wn1_ref,
             b_ref, o_ref):
    r = 1.0 / (deg_ref[...] + 1.0)
    acc = jnp.dot(x_ref[...], ws_ref[...], preferred_element_type=jnp.float32)
    acc += jnp.dot(a0_ref[...] * r, wn0_ref[...],
                   preferred_element_type=jnp.float32)
    acc += jnp.dot(a1_ref[...] * r, wn1_ref[...],
                   preferred_element_type=jnp.float32)
    o_ref[...] = acc + b_ref[...]


def _tc_combine(x, a0, a1, deg, w_self, w_neigh, b):
    wn0 = w_neigh[:_HALF]
    wn1 = w_neigh[_HALF:]
    deg2 = deg.reshape(_NPAD)[:_N].reshape(_N, 1)
    b2 = b.reshape(1, _D)
    return pl.pallas_call(
        _tc_body,
        grid=(_N // _BM,),
        in_specs=[
            pl.BlockSpec((_BM, _D), lambda i: (i, 0)),
            pl.BlockSpec((_BM, _HALF), lambda i: (i, 0)),
            pl.BlockSpec((_BM, _HALF), lambda i: (i, 0)),
            pl.BlockSpec((_BM, 1), lambda i: (i, 0)),
            pl.BlockSpec((_D, _D), lambda i: (0, 0)),
            pl.BlockSpec((_HALF, _D), lambda i: (0, 0)),
            pl.BlockSpec((_HALF, _D), lambda i: (0, 0)),
            pl.BlockSpec((1, _D), lambda i: (0, 0)),
        ],
        out_specs=pl.BlockSpec((_BM, _D), lambda i: (i, 0)),
        out_shape=jax.ShapeDtypeStruct((_N, _D), jnp.float32),
    )(x, a0, a1, deg2, w_self, wn0, wn1, b2)


def kernel(in_feat, edge_index, W_self, W_neigh, b):
    x_lo = in_feat[:, :_HALF]
    x_hi = in_feat[:, _HALF:]
    pad = jnp.full((_EPAD - _E,), _N, jnp.int32)
    src4 = jnp.concatenate(
        [edge_index[0], jnp.zeros((_EPAD - _E,), jnp.int32)]
    ).reshape(_TILES, _NB, 1, _K)
    dst4 = jnp.concatenate([edge_index[1], pad]).reshape(_TILES, _NB, 1, _K)
    agg_lo, agg_hi, deg = _sc_aggregate(x_lo, x_hi, src4, dst4)
    return _tc_combine(in_feat, agg_lo, agg_hi, deg, W_self, W_neigh, b)
